# baseline (speedup 1.0000x reference)
import functools

import jax
import jax.numpy as jnp
from jax import lax
from jax.experimental import pallas as pl
from jax.experimental.pallas import tpu as pltpu
from jax.experimental.pallas import tpu_sc as plsc

KNN = 20
EPSILON = 1e-5
ROW_TILE = 256
PT_TILE = 128
NEG_BIG = -1e30
NUM_WORKERS = 32


def _stage_a_body(n_pts, xr_ref, xa_ref, wa_ref, b_ref, idx_ref, y_ref):
    xr = xr_ref[0]
    xa = xa_ref[0]
    xrb = xr.astype(jnp.bfloat16)
    xab = xa.astype(jnp.bfloat16)
    dot = lax.dot_general(xrb, xab, (((1,), (1,)), ((), ())),
                          preferred_element_type=jnp.float32)
    sq_r = jnp.sum(xr * xr, axis=1, keepdims=True)
    sq_a = jnp.sum(xa * xa, axis=1)[None, :]
    d = 2.0 * dot - sq_r - sq_a
    n_iota = lax.broadcasted_iota(jnp.int32, d.shape, 1)
    cols = []
    for _ in range(KNN):
        m = jnp.max(d, axis=1, keepdims=True)
        cand = jnp.where(d == m, n_iota, n_pts)
        jmin = jnp.min(cand, axis=1)
        cols.append(jmin[None, :])
        d = jnp.where(n_iota == jmin[:, None], NEG_BIG, d)
    idx_ref[0] = jnp.concatenate(cols, axis=0)
    y_ref[0] = lax.dot_general(
        xrb, wa_ref[...].astype(jnp.bfloat16), (((1,), (0,)), ((), ())),
        preferred_element_type=jnp.float32) + b_ref[...]


def _stage_a(xt, wa, bias):
    bsz, n_pts, c_in = xt.shape
    g_out = wa.shape[1]
    r = ROW_TILE
    return pl.pallas_call(
        functools.partial(_stage_a_body, n_pts),
        grid=(bsz, n_pts // r),
        in_specs=[
            pl.BlockSpec((1, r, c_in), lambda b, t: (b, t, 0)),
            pl.BlockSpec((1, n_pts, c_in), lambda b, t: (b, 0, 0)),
            pl.BlockSpec((c_in, g_out), lambda b, t: (0, 0)),
            pl.BlockSpec((1, g_out), lambda b, t: (0, 0)),
        ],
        out_specs=[
            pl.BlockSpec((1, KNN, r), lambda b, t: (b, 0, t)),
            pl.BlockSpec((1, r, g_out), lambda b, t: (b, t, 0)),
        ],
        out_shape=[
            jax.ShapeDtypeStruct((bsz, KNN, n_pts), jnp.int32),
            jax.ShapeDtypeStruct((bsz, n_pts, g_out), jnp.float32),
        ],
    )(xt, xt, wa, bias)


def _make_stage_b(n_total, c_pad):
    rows = (n_total * KNN) // NUM_WORKERS
    nch = rows // 128
    mesh = plsc.VectorSubcoreMesh(core_axis_name="c", subcore_axis_name="s")

    @functools.partial(
        pl.kernel,
        out_type=jax.ShapeDtypeStruct((n_total * KNN, c_pad), jnp.float32),
        mesh=mesh,
        compiler_params=pltpu.CompilerParams(needs_layout_passes=False,
                                             use_tc_tiling_on_sc=False),
        scratch_types=[
            pltpu.VMEM((rows,), jnp.int32),
            pltpu.VMEM((128, c_pad), jnp.float32),
            pltpu.VMEM((128, c_pad), jnp.float32),
            pltpu.SemaphoreType.DMA,
            pltpu.SemaphoreType.DMA,
        ],
    )
    def stage_b(x_hbm, idx_hbm, xj_hbm, idx_v, buf0, buf1, sem0, sem1):
        wid = lax.axis_index("s") * 2 + lax.axis_index("c")
        base = wid * rows
        pltpu.sync_copy(idx_hbm.at[wid], idx_v)
        bufs = (buf0, buf1)
        sems = (sem0, sem1)
        copies = [None, None]
        for ch in range(nch + 1):
            if ch < nch:
                copies[ch % 2] = pltpu.async_copy(
                    x_hbm.at[idx_v.at[pl.ds(ch * 128, 128)]],
                    bufs[ch % 2], sems[ch % 2])
            if ch > 0:
                p = (ch - 1) % 2
                copies[p].wait()
                pltpu.sync_copy(
                    bufs[p], xj_hbm.at[pl.ds(base + (ch - 1) * 128, 128)])

    return stage_b


def _stage_c_body(xj_ref, xi_ref, wb_ref, mg_ref, s1_ref, s2_ref):
    pt = xi_ref.shape[0]
    c_pad = xi_ref.shape[1]
    xj = xj_ref[...].reshape(pt, KNN, c_pad)
    diff = (xj - xi_ref[...][:, None, :]).astype(jnp.bfloat16)
    wb = wb_ref[...].astype(jnp.bfloat16)
    gm = lax.dot_general(diff.reshape(pt * KNN, c_pad), wb,
                         (((1,), (0,)), ((), ())),
                         preferred_element_type=jnp.float32)
    gm = gm.reshape(pt, KNN, wb.shape[1])
    mg_ref[...] = jnp.max(gm, axis=1)
    s1_ref[...] = jnp.sum(gm, axis=1)
    s2_ref[...] = jnp.sum(gm * gm, axis=1)


def _stage_c(xj, x2, wb_pad):
    n_total, c_pad = x2.shape
    g_out = wb_pad.shape[1]
    pt = PT_TILE
    return pl.pallas_call(
        _stage_c_body,
        grid=(n_total // pt,),
        in_specs=[
            pl.BlockSpec((pt * KNN, c_pad), lambda t: (t, 0)),
            pl.BlockSpec((pt, c_pad), lambda t: (t, 0)),
            pl.BlockSpec((c_pad, g_out), lambda t: (0, 0)),
        ],
        out_specs=[
            pl.BlockSpec((pt, g_out), lambda t: (t, 0)),
            pl.BlockSpec((pt, g_out), lambda t: (t, 0)),
            pl.BlockSpec((pt, g_out), lambda t: (t, 0)),
        ],
        out_shape=[jax.ShapeDtypeStruct((n_total, g_out), jnp.float32)] * 3,
    )(xj, x2, wb_pad)


def _stage_d_body(y_ref, m_ref, s1_ref, s2_ref, g_ref, be_ref, f_ref):
    y = y_ref[...]
    s1 = s1_ref[...]
    k = float(KNN)
    cnt = k * y.shape[0]
    mean = jnp.sum(k * y + s1, axis=0, keepdims=True) / cnt
    e2 = jnp.sum(k * y * y + 2.0 * y * s1 + s2_ref[...], axis=0,
                 keepdims=True) / cnt
    var = e2 - mean * mean
    rstd = lax.rsqrt(var + EPSILON)
    f_ref[...] = jnp.maximum(
        (y + m_ref[...] - mean) * (rstd * g_ref[...]) + be_ref[...], 0.0)


def _stage_d(y2, m, s1, s2, gamma, beta):
    n_total, g_out = y2.shape
    return pl.pallas_call(
        _stage_d_body,
        out_shape=jax.ShapeDtypeStruct((n_total, g_out), jnp.float32),
    )(y2, m, s1, s2, gamma, beta)


def _edge_block(xt, w, bias, gamma, beta):
    bsz, n_pts, c_in = xt.shape
    g_out = w.shape[0]
    n_total = bsz * n_pts
    c_pad = c_in
    wa = jnp.transpose(w[:, :c_in], (1, 0))
    wb_pad = jnp.transpose(w[:, c_in:], (1, 0))
    idx, y = _stage_a(xt, wa, bias[None, :])

    boff = (jnp.arange(bsz, dtype=jnp.int32) * n_pts)[:, None, None]
    idx_g = (jnp.transpose(idx, (0, 2, 1)) + boff).reshape(
        NUM_WORKERS, (n_total * KNN) // NUM_WORKERS)
    x2 = xt.reshape(n_total, c_in)
    xj = _make_stage_b(n_total, c_pad)(x2, idx_g)
    mg, s1, s2 = _stage_c(xj, x2, wb_pad)
    f = _stage_d(y.reshape(n_total, g_out), mg, s1, s2,
                 gamma[None, :], beta[None, :])
    return f.reshape(bsz, n_pts, g_out)


def kernel(inputs, W0, b0, g0, be0, W1, b1, g1, be1, W2, b2, g2, be2):
    xt0 = jnp.transpose(inputs[..., 0], (0, 2, 1))
    f0 = _edge_block(xt0, W0, b0, g0, be0)
    c1 = jnp.concatenate([f0, xt0], axis=-1)
    f1 = _edge_block(c1, W1, b1, g1, be1)
    c2 = jnp.concatenate([f1, c1], axis=-1)
    f2 = _edge_block(c2, W2, b2, g2, be2)
    out = jnp.concatenate([f2, c2], axis=-1)
    return jnp.transpose(out, (0, 2, 1))[..., None]


# speedup vs baseline: 1.0502x; 1.0502x over previous
import functools

import jax
import jax.numpy as jnp
from jax import lax
from jax.experimental import pallas as pl
from jax.experimental.pallas import tpu as pltpu
from jax.experimental.pallas import tpu_sc as plsc

KNN = 20
EPSILON = 1e-5
ROW_TILE = 256
PT_TILE = 128
NEG_BIG = -1e30
NUM_WORKERS = 32


def _stage_a_body(n_pts, xr_ref, xa_ref, wa_ref, b_ref, idx_ref, y_ref):
    xr = xr_ref[0]
    xa = xa_ref[0]
    xrb = xr.astype(jnp.bfloat16)
    xab = xa.astype(jnp.bfloat16)
    dot = lax.dot_general(xrb, xab, (((1,), (1,)), ((), ())),
                          preferred_element_type=jnp.float32)
    sq_r = jnp.sum(xr * xr, axis=1, keepdims=True)
    sq_a = jnp.sum(xa * xa, axis=1)[None, :]
    d = 2.0 * dot - sq_r - sq_a
    n_iota = lax.broadcasted_iota(jnp.int32, d.shape, 1)
    cols = []
    for _ in range(KNN):
        m = jnp.max(d, axis=1, keepdims=True)
        cand = jnp.where(d == m, n_iota, n_pts)
        jmin = jnp.min(cand, axis=1)
        cols.append(jmin[None, :])
        d = jnp.where(n_iota == jmin[:, None], NEG_BIG, d)
    idx_ref[0] = jnp.concatenate(cols, axis=0)
    y_ref[0] = lax.dot_general(
        xrb, wa_ref[...].astype(jnp.bfloat16), (((1,), (0,)), ((), ())),
        preferred_element_type=jnp.float32) + b_ref[...]


def _stage_a(xt, wa, bias, half, n_half):
    bsz, n_pts, c_in = xt.shape
    g_out = wa.shape[1]
    r = ROW_TILE
    t0 = (half * n_half) // r
    return pl.pallas_call(
        functools.partial(_stage_a_body, n_pts),
        grid=(bsz, n_half // r),
        in_specs=[
            pl.BlockSpec((1, r, c_in), lambda b, t: (b, t0 + t, 0)),
            pl.BlockSpec((1, n_pts, c_in), lambda b, t: (b, 0, 0)),
            pl.BlockSpec((c_in, g_out), lambda b, t: (0, 0)),
            pl.BlockSpec((1, g_out), lambda b, t: (0, 0)),
        ],
        out_specs=[
            pl.BlockSpec((1, KNN, r), lambda b, t: (b, 0, t)),
            pl.BlockSpec((1, r, g_out), lambda b, t: (b, t, 0)),
        ],
        out_shape=[
            jax.ShapeDtypeStruct((bsz, KNN, n_half), jnp.int32),
            jax.ShapeDtypeStruct((bsz, n_half, g_out), jnp.float32),
        ],
    )(xt, xt, wa, bias)


def _make_stage_b(n_total, c_pad):
    rows = (n_total * KNN) // NUM_WORKERS
    nch = rows // 128
    mesh = plsc.VectorSubcoreMesh(core_axis_name="c", subcore_axis_name="s")

    @functools.partial(
        pl.kernel,
        out_type=jax.ShapeDtypeStruct((n_total * KNN, c_pad), jnp.float32),
        mesh=mesh,
        compiler_params=pltpu.CompilerParams(needs_layout_passes=False),
        scratch_types=[
            pltpu.VMEM((rows,), jnp.int32),
            pltpu.VMEM((128, c_pad), jnp.float32),
            pltpu.VMEM((128, c_pad), jnp.float32),
            pltpu.SemaphoreType.DMA,
            pltpu.SemaphoreType.DMA,
        ],
    )
    def stage_b(x_hbm, idx_hbm, xj_hbm, idx_v, buf0, buf1, sem0, sem1):
        wid = lax.axis_index("s") * 2 + lax.axis_index("c")
        base = wid * rows
        pltpu.sync_copy(idx_hbm.at[wid], idx_v)
        bufs = (buf0, buf1)
        sems = (sem0, sem1)
        copies = [None, None]
        for ch in range(nch + 1):
            if ch < nch:
                copies[ch % 2] = pltpu.async_copy(
                    x_hbm.at[idx_v.at[pl.ds(ch * 128, 128)]],
                    bufs[ch % 2], sems[ch % 2])
            if ch > 0:
                p = (ch - 1) % 2
                copies[p].wait()
                pltpu.sync_copy(
                    bufs[p], xj_hbm.at[pl.ds(base + (ch - 1) * 128, 128)])

    return stage_b


def _stage_c_body(xj_ref, xi_ref, wb_ref, mg_ref, s1_ref, s2_ref):
    pt = xi_ref.shape[0]
    c_pad = xi_ref.shape[1]
    xj = xj_ref[...].reshape(pt, KNN, c_pad)
    diff = (xj - xi_ref[...][:, None, :]).astype(jnp.bfloat16)
    wb = wb_ref[...].astype(jnp.bfloat16)
    gm = lax.dot_general(diff.reshape(pt * KNN, c_pad), wb,
                         (((1,), (0,)), ((), ())),
                         preferred_element_type=jnp.float32)
    gm = gm.reshape(pt, KNN, wb.shape[1])
    mg_ref[...] = jnp.max(gm, axis=1)
    s1_ref[...] = jnp.sum(gm, axis=1)
    s2_ref[...] = jnp.sum(gm * gm, axis=1)


def _stage_c(xj, x2, wb_pad, half, n_half, n_pts):
    _, c_pad = x2.shape
    bsz = x2.shape[0] // n_pts
    g_out = wb_pad.shape[1]
    pt = PT_TILE
    tpb = n_half // pt

    def xi_map(t):
        b = t // tpb
        return ((b * n_pts + half * n_half) // pt + t % tpb, 0)

    return pl.pallas_call(
        _stage_c_body,
        grid=(bsz * n_half // pt,),
        in_specs=[
            pl.BlockSpec((pt * KNN, c_pad), lambda t: (t, 0)),
            pl.BlockSpec((pt, c_pad), xi_map),
            pl.BlockSpec((c_pad, g_out), lambda t: (0, 0)),
        ],
        out_specs=[
            pl.BlockSpec((pt, g_out), lambda t: (t, 0)),
            pl.BlockSpec((pt, g_out), lambda t: (t, 0)),
            pl.BlockSpec((pt, g_out), lambda t: (t, 0)),
        ],
        out_shape=[jax.ShapeDtypeStruct((bsz * n_half, g_out),
                                        jnp.float32)] * 3,
    )(xj, x2, wb_pad)


def _stage_d_body(y_ref, m_ref, s1_ref, s2_ref, g_ref, be_ref, f_ref):
    y = y_ref[...]
    s1 = s1_ref[...]
    k = float(KNN)
    cnt = k * y.shape[0]
    mean = jnp.sum(k * y + s1, axis=0, keepdims=True) / cnt
    e2 = jnp.sum(k * y * y + 2.0 * y * s1 + s2_ref[...], axis=0,
                 keepdims=True) / cnt
    var = e2 - mean * mean
    rstd = lax.rsqrt(var + EPSILON)
    f_ref[...] = jnp.maximum(
        (y + m_ref[...] - mean) * (rstd * g_ref[...]) + be_ref[...], 0.0)


def _stage_d(y2, m, s1, s2, gamma, beta):
    n_total, g_out = y2.shape
    return pl.pallas_call(
        _stage_d_body,
        out_shape=jax.ShapeDtypeStruct((n_total, g_out), jnp.float32),
    )(y2, m, s1, s2, gamma, beta)


def _edge_block(xt, w, bias, gamma, beta):
    bsz, n_pts, c_in = xt.shape
    g_out = w.shape[0]
    n_total = bsz * n_pts
    n_half = n_pts // 2
    c_pad = ((c_in + 127) // 128) * 128
    wa = jnp.transpose(w[:, :c_in], (1, 0))
    wb = jnp.transpose(w[:, c_in:], (1, 0))
    wb_pad = jnp.concatenate(
        [wb, jnp.zeros((c_pad - c_in, g_out), jnp.float32)], axis=0)
    x2 = jnp.concatenate(
        [xt.reshape(n_total, c_in),
         jnp.zeros((n_total, c_pad - c_in), jnp.float32)], axis=1)
    boff = (jnp.arange(bsz, dtype=jnp.int32) * n_pts)[:, None, None]

    def half_graph(h):
        idx, y = _stage_a(xt, wa, bias[None, :], h, n_half)
        idx_g = (jnp.transpose(idx, (0, 2, 1)) + boff).reshape(
            NUM_WORKERS, (bsz * n_half * KNN) // NUM_WORKERS)
        xj = _make_stage_b(bsz * n_half, c_pad)(x2, idx_g)
        return y, xj

    y0, xj0 = half_graph(0)
    y1, xj1 = half_graph(1)
    mg0, s10, s20 = _stage_c(xj0, x2, wb_pad, 0, n_half, n_pts)
    mg1, s11, s21 = _stage_c(xj1, x2, wb_pad, 1, n_half, n_pts)

    def merge(a0, a1):
        return jnp.concatenate(
            [a0.reshape(bsz, n_half, g_out), a1.reshape(bsz, n_half, g_out)],
            axis=1).reshape(n_total, g_out)

    f = _stage_d(merge(y0, y1), merge(mg0, mg1), merge(s10, s11),
                 merge(s20, s21), gamma[None, :], beta[None, :])
    return f.reshape(bsz, n_pts, g_out)


def kernel(inputs, W0, b0, g0, be0, W1, b1, g1, be1, W2, b2, g2, be2):
    xt0 = jnp.transpose(inputs[..., 0], (0, 2, 1))
    f0 = _edge_block(xt0, W0, b0, g0, be0)
    c1 = jnp.concatenate([f0, xt0], axis=-1)
    f1 = _edge_block(c1, W1, b1, g1, be1)
    c2 = jnp.concatenate([f1, c1], axis=-1)
    f2 = _edge_block(c2, W2, b2, g2, be2)
    out = jnp.concatenate([f2, c2], axis=-1)
    return jnp.transpose(out, (0, 2, 1))[..., None]
